# baseline (speedup 1.0000x reference)
import functools

import jax
import jax.numpy as jnp
from jax import lax
from jax.experimental import pallas as pl
from jax.experimental.pallas import tpu as pltpu
from jax.experimental.pallas import tpu_sc as plsc

_EMBED_DIM = 32
_NUM_CORES = 2
_NUM_SUBCORES = 16
_NW = _NUM_CORES * _NUM_SUBCORES


def _make_gather(total, dim):
    b_per_w = total // _NW
    chunk = 3200
    nchunk = b_per_w // chunk

    mesh = plsc.VectorSubcoreMesh(core_axis_name="c", subcore_axis_name="s")

    @functools.partial(
        pl.kernel,
        mesh=mesh,
        out_type=jax.ShapeDtypeStruct((total, dim), jnp.float32),
        scratch_types=[
            pltpu.VMEM((chunk,), jnp.int32),
            pltpu.VMEM((chunk, dim), jnp.float32),
            pltpu.SemaphoreType.DMA,
        ],
        compiler_params=pltpu.CompilerParams(use_tc_tiling_on_sc=False),
    )
    def gather_kernel(idx_hbm, table_hbm, out_hbm, idx_v, rows_v, sem):
        wid = lax.axis_index("s") * _NUM_CORES + lax.axis_index("c")
        base = wid * b_per_w
        for i in range(nchunk):
            off = base + i * chunk
            pltpu.sync_copy(idx_hbm.at[pl.ds(off, chunk)], idx_v)
            pltpu.async_copy(table_hbm.at[idx_v], rows_v, sem).wait()
            pltpu.sync_copy(rows_v, out_hbm.at[pl.ds(off, chunk)])

    return gather_kernel


def kernel(indices, table):
    batch, hist = indices.shape
    idx_flat = indices.reshape(-1).astype(jnp.int32)
    out = _make_gather(batch * hist, _EMBED_DIM)(idx_flat, table)
    return out.reshape(batch, hist, _EMBED_DIM)


# speedup vs baseline: 1.0001x; 1.0001x over previous
import functools

import jax
import jax.numpy as jnp
from jax import lax
from jax.experimental import pallas as pl
from jax.experimental.pallas import tpu as pltpu
from jax.experimental.pallas import tpu_sc as plsc

_EMBED_DIM = 32
_NUM_CORES = 2
_NUM_SUBCORES = 16
_NW = _NUM_CORES * _NUM_SUBCORES

_CHUNK = 800


def _make_gather(total, dim):
    b_per_w = total // _NW
    chunk = _CHUNK
    nchunk = b_per_w // chunk
    nbuf = 4
    lag = 2

    mesh = plsc.VectorSubcoreMesh(core_axis_name="c", subcore_axis_name="s")

    @functools.partial(
        pl.kernel,
        mesh=mesh,
        out_type=jax.ShapeDtypeStruct((total, dim), jnp.float32),
        scratch_types=[
            pltpu.VMEM((nchunk, chunk), jnp.int32),
            [pltpu.VMEM((chunk, dim), jnp.float32) for _ in range(nbuf)],
            [pltpu.SemaphoreType.DMA for _ in range(nbuf)],
            [pltpu.SemaphoreType.DMA for _ in range(nbuf)],
        ],
        compiler_params=pltpu.CompilerParams(use_tc_tiling_on_sc=False),
    )
    def gather_kernel(idx_hbm, table_hbm, out_hbm, idx_v, rows, gsem, ssem):
        wid = lax.axis_index("s") * _NUM_CORES + lax.axis_index("c")
        base = wid * b_per_w
        pltpu.sync_copy(idx_hbm.at[pl.ds(wid * nchunk, nchunk)], idx_v)

        def fire_gather(i):
            b = i % nbuf
            pltpu.async_copy(table_hbm.at[idx_v.at[i]], rows[b], gsem[b])

        def retire(i):
            b = i % nbuf
            pltpu.make_async_copy(table_hbm.at[idx_v.at[i]], rows[b], gsem[b]).wait()
            pltpu.async_copy(
                rows[b], out_hbm.at[pl.ds(base + i * chunk, chunk)], ssem[b]
            )

        for i in range(nchunk):
            b = i % nbuf
            if i >= nbuf:
                pltpu.make_async_copy(
                    rows[b], out_hbm.at[pl.ds(base + (i - nbuf) * chunk, chunk)],
                    ssem[b],
                ).wait()
            fire_gather(i)
            if i >= lag:
                retire(i - lag)
        for i in range(nchunk - lag, nchunk):
            retire(i)
        for i in range(nchunk - nbuf, nchunk):
            b = i % nbuf
            pltpu.make_async_copy(
                rows[b], out_hbm.at[pl.ds(base + i * chunk, chunk)], ssem[b]
            ).wait()

    return gather_kernel


def kernel(indices, table):
    batch, hist = indices.shape
    total = batch * hist
    idx_2d = indices.reshape(total // _CHUNK, _CHUNK).astype(jnp.int32)
    out = _make_gather(total, _EMBED_DIM)(idx_2d, table)
    return out.reshape(batch, hist, _EMBED_DIM)
